# baseline (speedup 1.0000x reference)
import functools

import jax
import jax.numpy as jnp
from jax import lax
from jax.experimental import pallas as pl
from jax.experimental.pallas import tpu as pltpu
from jax.experimental.pallas import tpu_sc as plsc

BATCH = 4096
SEQ = 200
DIM = 64
N = BATCH * SEQ
IDX_MINOR = 128
NROWS = N // IDX_MINOR
NC = 2
NS = 16
NW = NC * NS
ROWS_PER_W = NROWS // NW
G = 4
N_CHUNKS = ROWS_PER_W // G


def _emb_body(x_hbm, table_hbm, out_hbm, idx_v, rows_v, sem):
    wid = lax.axis_index("s") * NC + lax.axis_index("c")
    row0 = wid * ROWS_PER_W

    def chunk(i, carry):
        r = row0 + i * G
        pltpu.sync_copy(x_hbm.at[pl.ds(r, G)], idx_v)
        copies = [
            pltpu.async_copy(
                table_hbm.at[idx_v.at[j]],
                rows_v.at[pl.ds(j * IDX_MINOR, IDX_MINOR)],
                sem,
            )
            for j in range(G)
        ]
        for c in copies:
            c.wait()
        pltpu.sync_copy(rows_v, out_hbm.at[pl.ds(r * IDX_MINOR, G * IDX_MINOR)])
        return carry

    lax.fori_loop(0, N_CHUNKS, chunk, 0)


def kernel(x, table):
    xf = x.reshape(NROWS, IDX_MINOR).astype(jnp.int32)
    mesh = plsc.VectorSubcoreMesh(core_axis_name="c", subcore_axis_name="s")
    out = pl.kernel(
        _emb_body,
        out_type=jax.ShapeDtypeStruct((N, DIM), jnp.float32),
        mesh=mesh,
        compiler_params=pltpu.CompilerParams(use_tc_tiling_on_sc=False),
        scratch_types=[
            pltpu.VMEM((G, IDX_MINOR), jnp.int32),
            pltpu.VMEM((G * IDX_MINOR, DIM), jnp.float32),
            pltpu.SemaphoreType.DMA,
        ],
    )(xf, table)
    return out.reshape(BATCH, SEQ, DIM)


# speedup vs baseline: 1.0371x; 1.0371x over previous
import jax
import jax.numpy as jnp
from jax import lax
from jax.experimental import pallas as pl
from jax.experimental.pallas import tpu as pltpu
from jax.experimental.pallas import tpu_sc as plsc

BATCH = 4096
SEQ = 200
DIM = 64
N = BATCH * SEQ
IDX_MINOR = 128
NROWS = N // IDX_MINOR
NC = 2
NS = 16
NW = NC * NS
ROWS_PER_W = NROWS // NW
G = 4
CHUNK = G * IDX_MINOR
N_CHUNKS = ROWS_PER_W // G
NBUF = 2


def _emb_body(x_hbm, table_hbm, out_hbm, idx_v, rows0, rows1, sem_g0, sem_g1,
              sem_w0, sem_w1):
    wid = lax.axis_index("s") * NC + lax.axis_index("c")
    row0 = wid * ROWS_PER_W
    rows = (rows0, rows1)
    sem_g = (sem_g0, sem_g1)
    sem_w = (sem_w0, sem_w1)

    pltpu.sync_copy(x_hbm.at[pl.ds(row0, ROWS_PER_W)], idx_v)

    def gather_desc(i, b, j):
        return pltpu.make_async_copy(
            table_hbm.at[idx_v.at[i * G + j]],
            rows[b].at[pl.ds(j * IDX_MINOR, IDX_MINOR)],
            sem_g[b],
        )

    def write_desc(i, b):
        return pltpu.make_async_copy(
            rows[b],
            out_hbm.at[pl.ds((row0 + i * G) * IDX_MINOR, CHUNK)],
            sem_w[b],
        )

    def fire_gather(i, b):
        for j in range(G):
            gather_desc(i, b, j).start()

    def drain_gather(i, b):
        for j in range(G):
            gather_desc(i, b, j).wait()

    fire_gather(0, 0)
    fire_gather(1, 1)
    drain_gather(0, 0)
    write_desc(0, 0).start()
    drain_gather(1, 1)
    write_desc(1, 1).start()

    def body(k, carry):
        for b in range(NBUF):
            i = 2 * k + b
            write_desc(i - NBUF, b).wait()
            fire_gather(i, b)
        for b in range(NBUF):
            i = 2 * k + b
            drain_gather(i, b)
            write_desc(i, b).start()
        return carry

    lax.fori_loop(1, N_CHUNKS // 2, body, 0)

    write_desc(N_CHUNKS - 2, 0).wait()
    write_desc(N_CHUNKS - 1, 1).wait()


def kernel(x, table):
    xf = x.reshape(NROWS, IDX_MINOR).astype(jnp.int32)
    mesh = plsc.VectorSubcoreMesh(core_axis_name="c", subcore_axis_name="s")
    out = pl.kernel(
        _emb_body,
        out_type=jax.ShapeDtypeStruct((N, DIM), jnp.float32),
        mesh=mesh,
        compiler_params=pltpu.CompilerParams(use_tc_tiling_on_sc=False),
        scratch_types=[
            pltpu.VMEM((ROWS_PER_W, IDX_MINOR), jnp.int32),
            pltpu.VMEM((CHUNK, DIM), jnp.float32),
            pltpu.VMEM((CHUNK, DIM), jnp.float32),
            pltpu.SemaphoreType.DMA,
            pltpu.SemaphoreType.DMA,
            pltpu.SemaphoreType.DMA,
            pltpu.SemaphoreType.DMA,
        ],
    )(xf, table)
    return out.reshape(BATCH, SEQ, DIM)
